# baseline (speedup 1.0000x reference)
import functools

import jax
import jax.numpy as jnp
from jax import lax
from jax.experimental import pallas as pl
from jax.experimental.pallas import tpu as pltpu
from jax.experimental.pallas import tpu_sc as plsc

VOCAB = 1000000
D = 64
B = 819200
NC = 2
NS = 16
NW = NC * NS
C = 128
NBLK = B // C
BLK_PER_W = NBLK // NW


def _sc_gather(idx2d, table):
    mesh = plsc.VectorSubcoreMesh(core_axis_name="c", subcore_axis_name="s")

    @functools.partial(
        pl.kernel,
        mesh=mesh,
        compiler_params=pltpu.CompilerParams(
            use_tc_tiling_on_sc=False, needs_layout_passes=False
        ),
        out_type=jax.ShapeDtypeStruct((D // 8, NBLK, 8 * C), jnp.float32),
        scratch_types=[
            pltpu.VMEM((BLK_PER_W, C), jnp.int32),
            [pltpu.VMEM((C, D), jnp.float32)] * 2,
            [pltpu.VMEM((D // 8, 8 * C), jnp.float32)] * 2,
            [pltpu.SemaphoreType.DMA] * 2,
            [pltpu.SemaphoreType.DMA] * 2,
        ],
    )
    def k(idx_hbm, table_hbm, out_hbm, idx_all, rows, outT, gsem, wsem):
        wid = lax.axis_index("s") * NC + lax.axis_index("c")
        blk0 = wid * BLK_PER_W
        pltpu.sync_copy(idx_hbm.at[pl.ds(blk0, BLK_PER_W)], idx_all)

        lane = lax.iota(jnp.int32, 16)
        tok_idx = [lane + 16 * g for g in range(C // 16)]
        zero16 = lane - lane

        def fire_gather(t, b):
            pltpu.async_copy(table_hbm.at[idx_all.at[t]], rows[b], gsem[b])

        def drain_gather(b):
            pltpu.make_async_copy(
                table_hbm.at[pl.ds(0, C)], rows[b], gsem[b]
            ).wait()

        def fire_write(t, b):
            pltpu.async_copy(outT[b], out_hbm.at[:, blk0 + t], wsem[b])

        def drain_write(b):
            pltpu.make_async_copy(outT[b], out_hbm.at[:, 0], wsem[b]).wait()

        def transpose_block(b):
            for d8 in range(D // 8):
                for u in range(8):
                    dvec = jnp.full((16,), d8 * 8 + u, jnp.int32)
                    for g in range(C // 16):
                        v = plsc.load_gather(rows[b], [tok_idx[g], dvec])
                        outT[b][d8, pl.ds(u * C + 16 * g, 16)] = v

        fire_gather(0, 0)
        fire_gather(1, 1)

        def body(i, carry):
            for u in range(2):
                t = 2 * i + u
                b = u
                drain_gather(b)

                @pl.when(i >= 1)
                def _():
                    drain_write(b)

                transpose_block(b)
                fire_write(t, b)

                @pl.when(i <= BLK_PER_W // 2 - 2)
                def _():
                    fire_gather(t + 2, b)

            return carry

        lax.fori_loop(0, BLK_PER_W // 2, body, 0)
        drain_write(0)
        drain_write(1)

    return k(idx2d, table)


def kernel(indices, table):
    idx2d = indices.astype(jnp.int32).reshape(NBLK, C)
    o3 = _sc_gather(idx2d, table)
    o4 = jnp.reshape(o3, (D // 8, NBLK, 8, C))
    return jnp.transpose(o4, (1, 3, 0, 2)).reshape(B, D)


# speedup vs baseline: 1.5857x; 1.5857x over previous
import functools

import jax
import jax.numpy as jnp
from jax import lax
from jax.experimental import pallas as pl
from jax.experimental.pallas import tpu as pltpu
from jax.experimental.pallas import tpu_sc as plsc

VOCAB = 1000000
D = 64
B = 819200
NC = 2
NS = 16
NW = NC * NS
C = 128
NBLK = B // C
BLK_PER_W = NBLK // NW


def _sc_gather(idx2d, table):
    mesh = plsc.VectorSubcoreMesh(core_axis_name="c", subcore_axis_name="s")

    @functools.partial(
        pl.kernel,
        mesh=mesh,
        compiler_params=pltpu.CompilerParams(
            use_tc_tiling_on_sc=False, needs_layout_passes=False
        ),
        out_type=jax.ShapeDtypeStruct((D // 8, NBLK, 8 * C), jnp.float32),
        scratch_types=[
            pltpu.VMEM((BLK_PER_W, C), jnp.int32),
            [pltpu.VMEM((C, D), jnp.float32)] * 2,
            [pltpu.VMEM((D // 8, 8 * C), jnp.float32)] * 2,
            [pltpu.SemaphoreType.DMA] * 2,
            [pltpu.SemaphoreType.DMA] * 2,
        ],
    )
    def k(idx_hbm, table_hbm, out_hbm, idx_all, rows, outT, gsem, wsem):
        wid = lax.axis_index("s") * NC + lax.axis_index("c")
        blk0 = wid * BLK_PER_W
        pltpu.sync_copy(idx_hbm.at[pl.ds(blk0, BLK_PER_W)], idx_all)

        lane = lax.iota(jnp.int32, 16)
        tok_idx = [lane + 16 * g for g in range(C // 16)]
        zero16 = lane - lane

        def fire_gather(t, b):
            pltpu.async_copy(table_hbm.at[idx_all.at[t]], rows[b], gsem[b])

        def drain_gather(b):
            pltpu.make_async_copy(
                table_hbm.at[pl.ds(0, C)], rows[b], gsem[b]
            ).wait()

        def fire_write(t, b):
            pltpu.async_copy(outT[b], out_hbm.at[:, blk0 + t], wsem[b])

        def drain_write(b):
            pltpu.make_async_copy(outT[b], out_hbm.at[:, 0], wsem[b]).wait()

        def transpose_block(b):
            @plsc.parallel_loop(0, D, step=1, unroll=8)
            def _(d):
                dvec = zero16 + d
                for g in range(C // 16):
                    v = plsc.load_gather(rows[b], [tok_idx[g], dvec])
                    outT[b][d // 8, pl.ds((d % 8) * C + 16 * g, 16)] = v

        fire_gather(0, 0)
        fire_gather(1, 1)

        def body(i, carry):
            for u in range(2):
                t = 2 * i + u
                b = u
                drain_gather(b)

                @pl.when(i >= 1)
                def _():
                    drain_write(b)

                transpose_block(b)
                fire_write(t, b)

                @pl.when(i <= BLK_PER_W // 2 - 2)
                def _():
                    fire_gather(t + 2, b)

            return carry

        lax.fori_loop(0, BLK_PER_W // 2, body, 0)
        drain_write(0)
        drain_write(1)

    return k(idx2d, table)


def kernel(indices, table):
    idx2d = indices.astype(jnp.int32).reshape(NBLK, C)
    o3 = _sc_gather(idx2d, table)
    o4 = jnp.reshape(o3, (D // 8, NBLK, 8, C))
    return jnp.transpose(o4, (1, 3, 0, 2)).reshape(B, D)


# speedup vs baseline: 1.5987x; 1.0082x over previous
import functools

import jax
import jax.numpy as jnp
from jax import lax
from jax.experimental import pallas as pl
from jax.experimental.pallas import tpu as pltpu
from jax.experimental.pallas import tpu_sc as plsc

VOCAB = 1000000
D = 64
B = 819200
NC = 2
NS = 16
NW = NC * NS
C = 128
NBLK = B // C
BLK_PER_W = NBLK // NW


def _sc_gather(idx2d, table):
    mesh = plsc.VectorSubcoreMesh(core_axis_name="c", subcore_axis_name="s")

    @functools.partial(
        pl.kernel,
        mesh=mesh,
        compiler_params=pltpu.CompilerParams(
            use_tc_tiling_on_sc=False, needs_layout_passes=False
        ),
        out_type=jax.ShapeDtypeStruct((D // 8, NBLK, 8 * C), jnp.float32),
        scratch_types=[
            pltpu.VMEM((BLK_PER_W, C), jnp.int32),
            [pltpu.VMEM((C, D), jnp.float32)] * 2,
            [pltpu.VMEM((D // 8, 8 * C), jnp.float32)] * 2,
            [pltpu.SemaphoreType.DMA] * 2,
            [pltpu.SemaphoreType.DMA] * 2,
        ],
    )
    def k(idx_hbm, table_hbm, out_hbm, idx_all, rows, outT, gsem, wsem):
        wid = lax.axis_index("s") * NC + lax.axis_index("c")
        blk0 = wid * BLK_PER_W
        pltpu.sync_copy(idx_hbm.at[pl.ds(blk0, BLK_PER_W)], idx_all)

        lane = lax.iota(jnp.int32, 16)
        tok_idx = [lane + 16 * g for g in range(C // 16)]
        zero16 = lane - lane

        def fire_gather(t, b):
            pltpu.async_copy(table_hbm.at[idx_all.at[t]], rows[b], gsem[b])

        def drain_gather(b):
            pltpu.make_async_copy(
                table_hbm.at[pl.ds(0, C)], rows[b], gsem[b]
            ).wait()

        def fire_write(t, b):
            pltpu.async_copy(outT[b], out_hbm.at[:, blk0 + t], wsem[b])

        def drain_write(b):
            pltpu.make_async_copy(outT[b], out_hbm.at[:, 0], wsem[b]).wait()

        def transpose_block(b):
            @plsc.parallel_loop(0, D, step=1, unroll=16)
            def _(d):
                dvec = zero16 + d
                for g in range(C // 16):
                    v = plsc.load_gather(rows[b], [tok_idx[g], dvec])
                    outT[b][d // 8, pl.ds((d % 8) * C + 16 * g, 16)] = v

        fire_gather(0, 0)
        fire_gather(1, 1)

        def body(i, carry):
            for u in range(2):
                t = 2 * i + u
                b = u
                drain_gather(b)

                @pl.when(i >= 1)
                def _():
                    drain_write(b)

                transpose_block(b)
                fire_write(t, b)

                @pl.when(i <= BLK_PER_W // 2 - 2)
                def _():
                    fire_gather(t + 2, b)

            return carry

        lax.fori_loop(0, BLK_PER_W // 2, body, 0)
        drain_write(0)
        drain_write(1)

    return k(idx2d, table)


def kernel(indices, table):
    idx2d = indices.astype(jnp.int32).reshape(NBLK, C)
    o3 = _sc_gather(idx2d, table)
    o4 = jnp.reshape(o3, (D // 8, NBLK, 8, C))
    return jnp.transpose(o4, (1, 3, 0, 2)).reshape(B, D)


# speedup vs baseline: 2.8064x; 1.7554x over previous
import functools

import jax
import jax.numpy as jnp
from jax import lax
from jax.experimental import pallas as pl
from jax.experimental.pallas import tpu as pltpu
from jax.experimental.pallas import tpu_sc as plsc

VOCAB = 1000000
HALF = VOCAB // 2
D = 64
B = 819200
NC = 2
NS = 16
NW = NC * NS
C = 128
P_PER_STEP = 128
P_PER_W = (B // 2) // NW
STEPS = P_PER_W // P_PER_STEP
NBUF = 4

W_IN = 4096
NB_IN = 123
P_ROWS = NB_IN * W_IN
VIEW_ROWS = 2 * P_ROWS
LAST_BLK = (VOCAB - 1) // W_IN


def _tc_to_token_major(t_feat):
    def body(lo_ref, hi_ref, o_ref):
        o_ref[:, 0:D] = jnp.transpose(lo_ref[...])
        o_ref[:, D : 2 * D] = jnp.transpose(hi_ref[...])

    return pl.pallas_call(
        body,
        grid=(NB_IN,),
        in_specs=[
            pl.BlockSpec((D, W_IN), lambda i: (0, jnp.minimum(2 * i, LAST_BLK))),
            pl.BlockSpec(
                (D, W_IN), lambda i: (0, jnp.minimum(2 * i + 1, LAST_BLK))
            ),
        ],
        out_specs=pl.BlockSpec((W_IN, 2 * D), lambda i: (i, 0)),
        out_shape=jax.ShapeDtypeStruct((P_ROWS, 2 * D), jnp.float32),
    )(t_feat, t_feat)


W_OUT = 4096


def _tc_to_feature_major(o_half):
    nb1 = (B // 2) // W_OUT

    def body(x_ref, o_ref):
        i = pl.program_id(0)

        @pl.when(i < nb1)
        def _():
            o_ref[...] = jnp.transpose(x_ref[:, 0:D])

        @pl.when(i >= nb1)
        def _():
            o_ref[...] = jnp.transpose(x_ref[:, D : 2 * D])

    return pl.pallas_call(
        body,
        grid=(2 * nb1,),
        in_specs=[
            pl.BlockSpec((W_OUT, 2 * D), lambda i: (i % nb1, 0)),
        ],
        out_specs=pl.BlockSpec((D, W_OUT), lambda i: (0, i)),
        out_shape=jax.ShapeDtypeStruct((D, B), jnp.float32),
    )(o_half)


def _sc_gather(idx_even, idx_odd, table_rows):
    mesh = plsc.VectorSubcoreMesh(core_axis_name="c", subcore_axis_name="s")

    @functools.partial(
        pl.kernel,
        mesh=mesh,
        compiler_params=pltpu.CompilerParams(use_tc_tiling_on_sc=False),
        out_type=jax.ShapeDtypeStruct((B // 2, 2 * D), jnp.float32),
        scratch_types=[
            pltpu.VMEM((STEPS, C), jnp.int32),
            pltpu.VMEM((STEPS, C), jnp.int32),
            [pltpu.VMEM((P_PER_STEP, D), jnp.float32)] * NBUF,
            [pltpu.VMEM((P_PER_STEP, D), jnp.float32)] * NBUF,
            [pltpu.SemaphoreType.DMA] * NBUF,
            [pltpu.SemaphoreType.DMA] * NBUF,
        ],
    )
    def k(
        idxe_hbm, idxo_hbm, table_hbm, out_hbm,
        idxe, idxo, rowse, rowso, gsem, wsem,
    ):
        wid = lax.axis_index("s") * NC + lax.axis_index("c")
        p0 = wid * P_PER_W
        pltpu.sync_copy(idxe_hbm.at[pl.ds(wid * STEPS, STEPS)], idxe)
        pltpu.sync_copy(idxo_hbm.at[pl.ds(wid * STEPS, STEPS)], idxo)

        def fire_gather(i, b):
            pltpu.async_copy(table_hbm.at[idxe.at[i]], rowse[b], gsem[b])
            pltpu.async_copy(table_hbm.at[idxo.at[i]], rowso[b], gsem[b])

        def drain_gather(b):
            pltpu.make_async_copy(
                table_hbm.at[pl.ds(0, P_PER_STEP)], rowse[b], gsem[b]
            ).wait()
            pltpu.make_async_copy(
                table_hbm.at[pl.ds(0, P_PER_STEP)], rowso[b], gsem[b]
            ).wait()

        def fire_write(i, b):
            r0 = p0 + i * P_PER_STEP
            pltpu.async_copy(
                rowse[b],
                out_hbm.at[pl.ds(r0, P_PER_STEP), pl.ds(0, D)],
                wsem[b],
            )
            pltpu.async_copy(
                rowso[b],
                out_hbm.at[pl.ds(r0, P_PER_STEP), pl.ds(D, D)],
                wsem[b],
            )

        def drain_write(b):
            pltpu.make_async_copy(
                rowse[b], out_hbm.at[pl.ds(0, P_PER_STEP), pl.ds(0, D)], wsem[b]
            ).wait()
            pltpu.make_async_copy(
                rowso[b], out_hbm.at[pl.ds(0, P_PER_STEP), pl.ds(D, D)], wsem[b]
            ).wait()

        fire_gather(0, 0)
        fire_gather(1, 1)
        drain_gather(0)
        fire_write(0, 0)
        fire_gather(2, 2)
        drain_gather(1)
        fire_write(1, 1)
        fire_gather(3, 3)

        def body(t, carry):
            base = 2 + t * 4
            for u in range(4):
                i = base + u
                b = (2 + u) % NBUF
                drain_gather(b)
                fire_write(i, b)
                drain_write(u % NBUF)
                fire_gather(i + 2, u % NBUF)
            return carry

        lax.fori_loop(0, (STEPS - 4) // 4, body, 0)

        drain_gather(2)
        fire_write(STEPS - 2, 2)
        drain_write(0)
        drain_gather(3)
        fire_write(STEPS - 1, 3)
        drain_write(1)
        drain_write(2)
        drain_write(3)

    return k(idx_even, idx_odd, table_rows)


def kernel(indices, table):
    idx = indices.astype(jnp.int32)
    g = idx // (2 * W_IN)
    m = idx % (2 * W_IN)
    base = g * W_IN
    row = jnp.where(m < W_IN, 2 * (base + m), 2 * (base + m - W_IN) + 1)
    idx_lo = row[: B // 2].reshape(B // (2 * C), C)
    idx_hi = row[B // 2 :].reshape(B // (2 * C), C)

    t_feat = jnp.transpose(table)
    table_rm = _tc_to_token_major(t_feat)
    o = _sc_gather(idx_lo, idx_hi, jnp.reshape(table_rm, (VIEW_ROWS, D)))
    o_feat = _tc_to_feature_major(o)
    return jnp.transpose(o_feat)


# speedup vs baseline: 3.1959x; 1.1388x over previous
import functools

import jax
import jax.numpy as jnp
from jax import lax
from jax.experimental import pallas as pl
from jax.experimental.pallas import tpu as pltpu
from jax.experimental.pallas import tpu_sc as plsc

VOCAB = 1000000
HALF = VOCAB // 2
D = 64
B = 819200
NC = 2
NS = 16
NW = NC * NS
C = 128
P_PER_STEP = 128
P_PER_W = (B // 2) // NW
STEPS = P_PER_W // P_PER_STEP
NBUF = 4

W_IN = 4096
NB_IN = 123
P_ROWS = NB_IN * W_IN
VIEW_ROWS = 2 * P_ROWS
LAST_BLK = (VOCAB - 1) // W_IN


def _tc_to_token_major(t_feat):
    def body(lo_ref, hi_ref, o_ref):
        o_ref[:, 0:D] = jnp.transpose(lo_ref[...])
        o_ref[:, D : 2 * D] = jnp.transpose(hi_ref[...])

    return pl.pallas_call(
        body,
        grid=(NB_IN,),
        in_specs=[
            pl.BlockSpec((D, W_IN), lambda i: (0, jnp.minimum(2 * i, LAST_BLK))),
            pl.BlockSpec(
                (D, W_IN), lambda i: (0, jnp.minimum(2 * i + 1, LAST_BLK))
            ),
        ],
        out_specs=pl.BlockSpec((W_IN, 2 * D), lambda i: (i, 0)),
        out_shape=jax.ShapeDtypeStruct((P_ROWS, 2 * D), jnp.float32),
    )(t_feat, t_feat)


W_OUT = 4096


def _tc_to_feature_major(o_half):
    def body(x_ref, o_ref):
        o_ref[:, 0:W_OUT] = jnp.transpose(x_ref[:, 0:D])
        o_ref[:, W_OUT : 2 * W_OUT] = jnp.transpose(x_ref[:, D : 2 * D])

    return pl.pallas_call(
        body,
        grid=(B // (2 * W_OUT),),
        in_specs=[
            pl.BlockSpec((W_OUT, 2 * D), lambda i: (i, 0)),
        ],
        out_specs=pl.BlockSpec((D, 2 * W_OUT), lambda i: (0, i)),
        out_shape=jax.ShapeDtypeStruct((D, B), jnp.float32),
    )(o_half)


def _sc_gather(idx_even, idx_odd, table_rows):
    mesh = plsc.VectorSubcoreMesh(core_axis_name="c", subcore_axis_name="s")

    @functools.partial(
        pl.kernel,
        mesh=mesh,
        compiler_params=pltpu.CompilerParams(use_tc_tiling_on_sc=False),
        out_type=jax.ShapeDtypeStruct((B // 2, 2 * D), jnp.float32),
        scratch_types=[
            pltpu.VMEM((STEPS, C), jnp.int32),
            pltpu.VMEM((STEPS, C), jnp.int32),
            [pltpu.VMEM((P_PER_STEP, D), jnp.float32)] * NBUF,
            [pltpu.VMEM((P_PER_STEP, D), jnp.float32)] * NBUF,
            [pltpu.SemaphoreType.DMA] * NBUF,
            [pltpu.SemaphoreType.DMA] * NBUF,
        ],
    )
    def k(
        idxe_hbm, idxo_hbm, table_hbm, out_hbm,
        idxe, idxo, rowse, rowso, gsem, wsem,
    ):
        wid = lax.axis_index("s") * NC + lax.axis_index("c")
        p0 = wid * P_PER_W
        pltpu.sync_copy(idxe_hbm.at[pl.ds(wid * STEPS, STEPS)], idxe)
        pltpu.sync_copy(idxo_hbm.at[pl.ds(wid * STEPS, STEPS)], idxo)

        def fire_gather(i, b):
            pltpu.async_copy(table_hbm.at[idxe.at[i]], rowse[b], gsem[b])
            pltpu.async_copy(table_hbm.at[idxo.at[i]], rowso[b], gsem[b])

        def drain_gather(b):
            pltpu.make_async_copy(
                table_hbm.at[pl.ds(0, P_PER_STEP)], rowse[b], gsem[b]
            ).wait()
            pltpu.make_async_copy(
                table_hbm.at[pl.ds(0, P_PER_STEP)], rowso[b], gsem[b]
            ).wait()

        def fire_write(i, b):
            r0 = p0 + i * P_PER_STEP
            pltpu.async_copy(
                rowse[b],
                out_hbm.at[pl.ds(r0, P_PER_STEP), pl.ds(0, D)],
                wsem[b],
            )
            pltpu.async_copy(
                rowso[b],
                out_hbm.at[pl.ds(r0, P_PER_STEP), pl.ds(D, D)],
                wsem[b],
            )

        def drain_write(b):
            pltpu.make_async_copy(
                rowse[b], out_hbm.at[pl.ds(0, P_PER_STEP), pl.ds(0, D)], wsem[b]
            ).wait()
            pltpu.make_async_copy(
                rowso[b], out_hbm.at[pl.ds(0, P_PER_STEP), pl.ds(D, D)], wsem[b]
            ).wait()

        fire_gather(0, 0)
        fire_gather(1, 1)
        drain_gather(0)
        fire_write(0, 0)
        fire_gather(2, 2)
        drain_gather(1)
        fire_write(1, 1)
        fire_gather(3, 3)

        def body(t, carry):
            base = 2 + t * 4
            for u in range(4):
                i = base + u
                b = (2 + u) % NBUF
                drain_gather(b)
                fire_write(i, b)
                drain_write(u % NBUF)
                fire_gather(i + 2, u % NBUF)
            return carry

        lax.fori_loop(0, (STEPS - 4) // 4, body, 0)

        drain_gather(2)
        fire_write(STEPS - 2, 2)
        drain_write(0)
        drain_gather(3)
        fire_write(STEPS - 1, 3)
        drain_write(1)
        drain_write(2)
        drain_write(3)

    return k(idx_even, idx_odd, table_rows)


def kernel(indices, table):
    idx = indices.astype(jnp.int32)
    g = idx // (2 * W_IN)
    m = idx % (2 * W_IN)
    base = g * W_IN
    row = jnp.where(m < W_IN, 2 * (base + m), 2 * (base + m - W_IN) + 1)
    rr = row.reshape(B // (2 * W_OUT), 2, W_OUT)
    idx_lo = rr[:, 0, :].reshape(B // (2 * C), C)
    idx_hi = rr[:, 1, :].reshape(B // (2 * C), C)

    t_feat = jnp.transpose(table)
    table_rm = _tc_to_token_major(t_feat)
    o = _sc_gather(idx_lo, idx_hi, jnp.reshape(table_rm, (VIEW_ROWS, D)))
    o_feat = _tc_to_feature_major(o)
    return jnp.transpose(o_feat)


# speedup vs baseline: 3.5228x; 1.1023x over previous
import functools

import jax
import jax.numpy as jnp
from jax import lax
from jax.experimental import pallas as pl
from jax.experimental.pallas import tpu as pltpu
from jax.experimental.pallas import tpu_sc as plsc

VOCAB = 1000000
HALF = VOCAB // 2
D = 64
B = 819200
NC = 2
NS = 16
NW = NC * NS
C = 128
P_PER_STEP = 128
P_PER_W = (B // 2) // NW
STEPS = P_PER_W // P_PER_STEP
NBUF = 4

W_IN = 8192
NB_IN = 62
P_ROWS = NB_IN * W_IN
VIEW_ROWS = 2 * P_ROWS
LAST_BLK = (VOCAB - 1) // W_IN


def _tc_to_token_major(t_feat):
    def body(lo_ref, hi_ref, o_ref):
        o_ref[:, 0:D] = jnp.transpose(lo_ref[...])
        o_ref[:, D : 2 * D] = jnp.transpose(hi_ref[...])

    return pl.pallas_call(
        body,
        grid=(NB_IN,),
        in_specs=[
            pl.BlockSpec((D, W_IN), lambda i: (0, jnp.minimum(2 * i, LAST_BLK))),
            pl.BlockSpec(
                (D, W_IN), lambda i: (0, jnp.minimum(2 * i + 1, LAST_BLK))
            ),
        ],
        out_specs=pl.BlockSpec((W_IN, 2 * D), lambda i: (i, 0)),
        out_shape=jax.ShapeDtypeStruct((P_ROWS, 2 * D), jnp.float32),
    )(t_feat, t_feat)


W_OUT = 8192


def _tc_to_feature_major(o_half):
    def body(x_ref, o_ref):
        o_ref[:, 0:W_OUT] = jnp.transpose(x_ref[:, 0:D])
        o_ref[:, W_OUT : 2 * W_OUT] = jnp.transpose(x_ref[:, D : 2 * D])

    return pl.pallas_call(
        body,
        grid=(B // (2 * W_OUT),),
        in_specs=[
            pl.BlockSpec((W_OUT, 2 * D), lambda i: (i, 0)),
        ],
        out_specs=pl.BlockSpec((D, 2 * W_OUT), lambda i: (0, i)),
        out_shape=jax.ShapeDtypeStruct((D, B), jnp.float32),
    )(o_half)


def _sc_gather(idx_even, idx_odd, table_rows):
    mesh = plsc.VectorSubcoreMesh(core_axis_name="c", subcore_axis_name="s")

    @functools.partial(
        pl.kernel,
        mesh=mesh,
        compiler_params=pltpu.CompilerParams(use_tc_tiling_on_sc=False),
        out_type=jax.ShapeDtypeStruct((B // 2, 2 * D), jnp.float32),
        scratch_types=[
            pltpu.VMEM((STEPS, C), jnp.int32),
            pltpu.VMEM((STEPS, C), jnp.int32),
            [pltpu.VMEM((P_PER_STEP, D), jnp.float32)] * NBUF,
            [pltpu.VMEM((P_PER_STEP, D), jnp.float32)] * NBUF,
            [pltpu.SemaphoreType.DMA] * NBUF,
            [pltpu.SemaphoreType.DMA] * NBUF,
        ],
    )
    def k(
        idxe_hbm, idxo_hbm, table_hbm, out_hbm,
        idxe, idxo, rowse, rowso, gsem, wsem,
    ):
        wid = lax.axis_index("s") * NC + lax.axis_index("c")
        p0 = wid * P_PER_W
        pltpu.sync_copy(idxe_hbm.at[pl.ds(wid * STEPS, STEPS)], idxe)
        pltpu.sync_copy(idxo_hbm.at[pl.ds(wid * STEPS, STEPS)], idxo)

        def fire_gather(i, b):
            pltpu.async_copy(table_hbm.at[idxe.at[i]], rowse[b], gsem[b])
            pltpu.async_copy(table_hbm.at[idxo.at[i]], rowso[b], gsem[b])

        def drain_gather(b):
            pltpu.make_async_copy(
                table_hbm.at[pl.ds(0, P_PER_STEP)], rowse[b], gsem[b]
            ).wait()
            pltpu.make_async_copy(
                table_hbm.at[pl.ds(0, P_PER_STEP)], rowso[b], gsem[b]
            ).wait()

        def fire_write(i, b):
            r0 = p0 + i * P_PER_STEP
            pltpu.async_copy(
                rowse[b],
                out_hbm.at[pl.ds(r0, P_PER_STEP), pl.ds(0, D)],
                wsem[b],
            )
            pltpu.async_copy(
                rowso[b],
                out_hbm.at[pl.ds(r0, P_PER_STEP), pl.ds(D, D)],
                wsem[b],
            )

        def drain_write(b):
            pltpu.make_async_copy(
                rowse[b], out_hbm.at[pl.ds(0, P_PER_STEP), pl.ds(0, D)], wsem[b]
            ).wait()
            pltpu.make_async_copy(
                rowso[b], out_hbm.at[pl.ds(0, P_PER_STEP), pl.ds(D, D)], wsem[b]
            ).wait()

        fire_gather(0, 0)
        fire_gather(1, 1)
        drain_gather(0)
        fire_write(0, 0)
        fire_gather(2, 2)
        drain_gather(1)
        fire_write(1, 1)
        fire_gather(3, 3)

        def body(t, carry):
            base = 2 + t * 4
            for u in range(4):
                i = base + u
                b = (2 + u) % NBUF
                drain_gather(b)
                fire_write(i, b)
                drain_write(u % NBUF)
                fire_gather(i + 2, u % NBUF)
            return carry

        lax.fori_loop(0, (STEPS - 4) // 4, body, 0)

        drain_gather(2)
        fire_write(STEPS - 2, 2)
        drain_write(0)
        drain_gather(3)
        fire_write(STEPS - 1, 3)
        drain_write(1)
        drain_write(2)
        drain_write(3)

    return k(idx_even, idx_odd, table_rows)


def kernel(indices, table):
    idx = indices.astype(jnp.int32)
    g = idx // (2 * W_IN)
    m = idx % (2 * W_IN)
    base = g * W_IN
    row = jnp.where(m < W_IN, 2 * (base + m), 2 * (base + m - W_IN) + 1)
    rr = row.reshape(B // (2 * W_OUT), 2, W_OUT)
    idx_lo = rr[:, 0, :].reshape(B // (2 * C), C)
    idx_hi = rr[:, 1, :].reshape(B // (2 * C), C)

    t_feat = jnp.transpose(table)
    table_rm = _tc_to_token_major(t_feat)
    o = _sc_gather(idx_lo, idx_hi, jnp.reshape(table_rm, (VIEW_ROWS, D)))
    o_feat = _tc_to_feature_major(o)
    return jnp.transpose(o_feat)


# speedup vs baseline: 3.6790x; 1.0443x over previous
import functools

import jax
import jax.numpy as jnp
from jax import lax
from jax.experimental import pallas as pl
from jax.experimental.pallas import tpu as pltpu
from jax.experimental.pallas import tpu_sc as plsc

VOCAB = 1000000
HALF = VOCAB // 2
D = 64
B = 819200
NC = 2
NS = 16
NW = NC * NS
C = 128
P_PER_STEP = 128
P_PER_W = (B // 2) // NW
STEPS = P_PER_W // P_PER_STEP
NBUF = 4

W_IN = 16384
NB_IN = 31
P_ROWS = NB_IN * W_IN
VIEW_ROWS = 2 * P_ROWS
LAST_BLK = (VOCAB - 1) // W_IN


def _tc_to_token_major(t_feat):
    def body(lo_ref, hi_ref, o_ref):
        o_ref[:, 0:D] = jnp.transpose(lo_ref[...])
        o_ref[:, D : 2 * D] = jnp.transpose(hi_ref[...])

    return pl.pallas_call(
        body,
        grid=(NB_IN,),
        in_specs=[
            pl.BlockSpec((D, W_IN), lambda i: (0, jnp.minimum(2 * i, LAST_BLK))),
            pl.BlockSpec(
                (D, W_IN), lambda i: (0, jnp.minimum(2 * i + 1, LAST_BLK))
            ),
        ],
        out_specs=pl.BlockSpec((W_IN, 2 * D), lambda i: (i, 0)),
        out_shape=jax.ShapeDtypeStruct((P_ROWS, 2 * D), jnp.float32),
    )(t_feat, t_feat)


W_OUT = 16384


def _tc_to_feature_major(o_half):
    def body(x_ref, o_ref):
        o_ref[:, 0:W_OUT] = jnp.transpose(x_ref[:, 0:D])
        o_ref[:, W_OUT : 2 * W_OUT] = jnp.transpose(x_ref[:, D : 2 * D])

    return pl.pallas_call(
        body,
        grid=(B // (2 * W_OUT),),
        in_specs=[
            pl.BlockSpec((W_OUT, 2 * D), lambda i: (i, 0)),
        ],
        out_specs=pl.BlockSpec((D, 2 * W_OUT), lambda i: (0, i)),
        out_shape=jax.ShapeDtypeStruct((D, B), jnp.float32),
    )(o_half)


def _sc_gather(idx_even, idx_odd, table_rows):
    mesh = plsc.VectorSubcoreMesh(core_axis_name="c", subcore_axis_name="s")

    @functools.partial(
        pl.kernel,
        mesh=mesh,
        compiler_params=pltpu.CompilerParams(use_tc_tiling_on_sc=False),
        out_type=jax.ShapeDtypeStruct((B // 2, 2 * D), jnp.float32),
        scratch_types=[
            pltpu.VMEM((STEPS, C), jnp.int32),
            pltpu.VMEM((STEPS, C), jnp.int32),
            [pltpu.VMEM((P_PER_STEP, D), jnp.float32)] * NBUF,
            [pltpu.VMEM((P_PER_STEP, D), jnp.float32)] * NBUF,
            [pltpu.SemaphoreType.DMA] * NBUF,
            [pltpu.SemaphoreType.DMA] * NBUF,
        ],
    )
    def k(
        idxe_hbm, idxo_hbm, table_hbm, out_hbm,
        idxe, idxo, rowse, rowso, gsem, wsem,
    ):
        wid = lax.axis_index("s") * NC + lax.axis_index("c")
        p0 = wid * P_PER_W
        pltpu.sync_copy(idxe_hbm.at[pl.ds(wid * STEPS, STEPS)], idxe)
        pltpu.sync_copy(idxo_hbm.at[pl.ds(wid * STEPS, STEPS)], idxo)

        def fire_gather(i, b):
            pltpu.async_copy(table_hbm.at[idxe.at[i]], rowse[b], gsem[b])
            pltpu.async_copy(table_hbm.at[idxo.at[i]], rowso[b], gsem[b])

        def drain_gather(b):
            pltpu.make_async_copy(
                table_hbm.at[pl.ds(0, P_PER_STEP)], rowse[b], gsem[b]
            ).wait()
            pltpu.make_async_copy(
                table_hbm.at[pl.ds(0, P_PER_STEP)], rowso[b], gsem[b]
            ).wait()

        def fire_write(i, b):
            r0 = p0 + i * P_PER_STEP
            pltpu.async_copy(
                rowse[b],
                out_hbm.at[pl.ds(r0, P_PER_STEP), pl.ds(0, D)],
                wsem[b],
            )
            pltpu.async_copy(
                rowso[b],
                out_hbm.at[pl.ds(r0, P_PER_STEP), pl.ds(D, D)],
                wsem[b],
            )

        def drain_write(b):
            pltpu.make_async_copy(
                rowse[b], out_hbm.at[pl.ds(0, P_PER_STEP), pl.ds(0, D)], wsem[b]
            ).wait()
            pltpu.make_async_copy(
                rowso[b], out_hbm.at[pl.ds(0, P_PER_STEP), pl.ds(D, D)], wsem[b]
            ).wait()

        fire_gather(0, 0)
        fire_gather(1, 1)
        drain_gather(0)
        fire_write(0, 0)
        fire_gather(2, 2)
        drain_gather(1)
        fire_write(1, 1)
        fire_gather(3, 3)

        def body(t, carry):
            base = 2 + t * 4
            for u in range(4):
                i = base + u
                b = (2 + u) % NBUF
                drain_gather(b)
                fire_write(i, b)
                drain_write(u % NBUF)
                fire_gather(i + 2, u % NBUF)
            return carry

        lax.fori_loop(0, (STEPS - 4) // 4, body, 0)

        drain_gather(2)
        fire_write(STEPS - 2, 2)
        drain_write(0)
        drain_gather(3)
        fire_write(STEPS - 1, 3)
        drain_write(1)
        drain_write(2)
        drain_write(3)

    return k(idx_even, idx_odd, table_rows)


def kernel(indices, table):
    idx = indices.astype(jnp.int32)
    g = idx // (2 * W_IN)
    m = idx % (2 * W_IN)
    base = g * W_IN
    row = jnp.where(m < W_IN, 2 * (base + m), 2 * (base + m - W_IN) + 1)
    rr = row.reshape(B // (2 * W_OUT), 2, W_OUT)
    idx_lo = rr[:, 0, :].reshape(B // (2 * C), C)
    idx_hi = rr[:, 1, :].reshape(B // (2 * C), C)

    t_feat = jnp.transpose(table)
    table_rm = _tc_to_token_major(t_feat)
    o = _sc_gather(idx_lo, idx_hi, jnp.reshape(table_rm, (VIEW_ROWS, D)))
    o_feat = _tc_to_feature_major(o)
    return jnp.transpose(o_feat)
